# Initial kernel scaffold; baseline (speedup 1.0000x reference)
import functools

import jax
import jax.numpy as jnp
from jax import lax
from jax.experimental import pallas as pl
from jax.experimental.pallas import tpu as pltpu
from jax.experimental.pallas import tpu_sc as plsc

ROWS = 16384
COLS = 16384
W = 164
NNZ = ROWS * W

NUM_WORKERS = 32
RPW = ROWS // NUM_WORKERS
CHUNK_ROWS = 128
CHUNK_NNZ = CHUNK_ROWS * W
N_CHUNKS = RPW // CHUNK_ROWS
STRIPS = CHUNK_ROWS // 16


def _spmv_body(x_hbm, vals_hbm, cols_hbm, y_hbm, x_v, vals_v, cols_v, y_v):
    wid = lax.axis_index("s") * 2 + lax.axis_index("c")

    pltpu.sync_copy(x_hbm, x_v)

    lane = lax.broadcasted_iota(jnp.int32, (16,), 0)
    stride_idx = lane * W

    def chunk_body(c, _):
        row0 = wid * RPW + c * CHUNK_ROWS
        nnz0 = row0 * W
        pltpu.sync_copy(vals_hbm.at[pl.ds(nnz0, CHUNK_NNZ)], vals_v)
        pltpu.sync_copy(cols_hbm.at[pl.ds(nnz0, CHUNK_NNZ)], cols_v)

        def strip_body(s, _):
            base = s * (16 * W)

            def j_body(j, acc):
                pos = stride_idx + (base + j)
                cv = plsc.load_gather(cols_v, [pos])
                vv = plsc.load_gather(vals_v, [pos])
                xg = plsc.load_gather(x_v, [cv])
                return acc + vv * xg

            acc = lax.fori_loop(0, W, j_body, jnp.zeros((16,), jnp.float32))
            y_v[pl.ds(c * CHUNK_ROWS + s * 16, 16)] = acc
            return 0

        lax.fori_loop(0, STRIPS, strip_body, 0)
        return 0

    lax.fori_loop(0, N_CHUNKS, chunk_body, 0)

    pltpu.sync_copy(y_v, y_hbm.at[pl.ds(wid * RPW, RPW)])


@jax.jit
def _spmv(x, values, cols_i32):
    mesh = plsc.VectorSubcoreMesh(core_axis_name="c", subcore_axis_name="s")
    return pl.kernel(
        _spmv_body,
        mesh=mesh,
        out_type=jax.ShapeDtypeStruct((ROWS,), jnp.float32),
        scratch_types=[
            pltpu.VMEM((COLS,), jnp.float32),
            pltpu.VMEM((CHUNK_NNZ,), jnp.float32),
            pltpu.VMEM((CHUNK_NNZ,), jnp.int32),
            pltpu.VMEM((RPW,), jnp.float32),
        ],
    )(x, values, cols_i32)


def kernel(x, values, col_indices, row_ptrs):
    del row_ptrs
    return _spmv(x, values, col_indices.astype(jnp.int32))


# speedup vs baseline: 6177.1110x; 6177.1110x over previous
import functools

import jax
import jax.numpy as jnp
from jax import lax
from jax.experimental import pallas as pl
from jax.experimental.pallas import tpu as pltpu
from jax.experimental.pallas import tpu_sc as plsc

ROWS = 16384
COLS = 16384
W = 164
NNZ = ROWS * W

NUM_WORKERS = 32
RPW = ROWS // NUM_WORKERS
CHUNK_ROWS = 128
CHUNK_NNZ = CHUNK_ROWS * W
N_CHUNKS = RPW // CHUNK_ROWS
STRIPS = CHUNK_ROWS // 16


def _spmv_body(x_hbm, vals_hbm, cols_hbm, y_hbm, x_v, vals_v, cols_v, y_v):
    wid = lax.axis_index("s") * 2 + lax.axis_index("c")

    pltpu.sync_copy(x_hbm, x_v)

    lane = lax.broadcasted_iota(jnp.int32, (16,), 0)
    stride_idx = lane * W

    def chunk_body(c, _):
        row0 = wid * RPW + c * CHUNK_ROWS
        nnz0 = row0 * W
        pltpu.sync_copy(vals_hbm.at[pl.ds(nnz0, CHUNK_NNZ)], vals_v)
        pltpu.sync_copy(cols_hbm.at[pl.ds(nnz0, CHUNK_NNZ)], cols_v)

        def strip_body(s, _):
            base = s * (16 * W)

            def j_body(j, acc):
                pos = stride_idx + (base + j)
                cv = plsc.load_gather(cols_v, [pos])
                vv = plsc.load_gather(vals_v, [pos])
                xg = plsc.load_gather(x_v, [cv])
                return acc + vv * xg

            acc = lax.fori_loop(0, W, j_body, jnp.zeros((16,), jnp.float32))
            y_v[pl.ds(c * CHUNK_ROWS + s * 16, 16)] = acc
            return 0

        lax.fori_loop(0, STRIPS, strip_body, 0)
        return 0

    lax.fori_loop(0, N_CHUNKS, chunk_body, 0)

    pltpu.sync_copy(y_v, y_hbm.at[pl.ds(wid * RPW, RPW)])


@jax.jit
def _spmv(x, values, cols_i32):
    mesh = plsc.VectorSubcoreMesh(core_axis_name="c", subcore_axis_name="s")
    return pl.kernel(
        _spmv_body,
        mesh=mesh,
        compiler_params=pltpu.CompilerParams(needs_layout_passes=False),
        out_type=jax.ShapeDtypeStruct((ROWS,), jnp.float32),
        scratch_types=[
            pltpu.VMEM((COLS,), jnp.float32),
            pltpu.VMEM((CHUNK_NNZ,), jnp.float32),
            pltpu.VMEM((CHUNK_NNZ,), jnp.int32),
            pltpu.VMEM((RPW,), jnp.float32),
        ],
    )(x, values, cols_i32)


def kernel(x, values, col_indices, row_ptrs):
    del row_ptrs
    return _spmv(x, values, col_indices.astype(jnp.int32))


# speedup vs baseline: 9167.6292x; 1.4841x over previous
import functools

import jax
import jax.numpy as jnp
from jax import lax
from jax.experimental import pallas as pl
from jax.experimental.pallas import tpu as pltpu
from jax.experimental.pallas import tpu_sc as plsc

ROWS = 16384
COLS = 16384
W = 164
NNZ = ROWS * W

NUM_WORKERS = 32
RPW = ROWS // NUM_WORKERS
CHUNK_ROWS = 128
CHUNK_NNZ = CHUNK_ROWS * W
N_CHUNKS = RPW // CHUNK_ROWS
STRIPS = CHUNK_ROWS // 16


def _spmv_body(x_hbm, vals_hbm, cols_hbm, y_hbm, x_v, vals_v, cols_v, y_v):
    wid = lax.axis_index("s") * 2 + lax.axis_index("c")

    pltpu.sync_copy(x_hbm, x_v)

    lane = lax.broadcasted_iota(jnp.int32, (16,), 0)
    stride_idx = lane * W

    def chunk_body(c, _):
        row0 = wid * RPW + c * CHUNK_ROWS
        nnz0 = row0 * W
        pltpu.sync_copy(vals_hbm.at[pl.ds(nnz0, CHUNK_NNZ)], vals_v)
        pltpu.sync_copy(cols_hbm.at[pl.ds(nnz0, CHUNK_NNZ)], cols_v)

        def strip_body(s, _):
            base = s * (16 * W)
            zero = jnp.zeros((16,), jnp.float32)

            @plsc.parallel_loop(0, W, step=4, carry=(zero, zero, zero, zero))
            def j_loop(j, accs):
                outs = []
                for u in range(4):
                    pos = stride_idx + (base + j + u)
                    cv = plsc.load_gather(cols_v, [pos])
                    vv = plsc.load_gather(vals_v, [pos])
                    xg = plsc.load_gather(x_v, [cv])
                    outs.append(accs[u] + vv * xg)
                return tuple(outs)

            a0, a1, a2, a3 = j_loop
            y_v[pl.ds(c * CHUNK_ROWS + s * 16, 16)] = (a0 + a1) + (a2 + a3)
            return 0

        lax.fori_loop(0, STRIPS, strip_body, 0)
        return 0

    lax.fori_loop(0, N_CHUNKS, chunk_body, 0)

    pltpu.sync_copy(y_v, y_hbm.at[pl.ds(wid * RPW, RPW)])


@jax.jit
def _spmv(x, values, cols_i32):
    mesh = plsc.VectorSubcoreMesh(core_axis_name="c", subcore_axis_name="s")
    return pl.kernel(
        _spmv_body,
        mesh=mesh,
        compiler_params=pltpu.CompilerParams(needs_layout_passes=False),
        out_type=jax.ShapeDtypeStruct((ROWS,), jnp.float32),
        scratch_types=[
            pltpu.VMEM((COLS,), jnp.float32),
            pltpu.VMEM((CHUNK_NNZ,), jnp.float32),
            pltpu.VMEM((CHUNK_NNZ,), jnp.int32),
            pltpu.VMEM((RPW,), jnp.float32),
        ],
    )(x, values, cols_i32)


def kernel(x, values, col_indices, row_ptrs):
    del row_ptrs
    return _spmv(x, values, col_indices.astype(jnp.int32))


# speedup vs baseline: 11396.3666x; 1.2431x over previous
import functools

import jax
import jax.numpy as jnp
from jax import lax
from jax.experimental import pallas as pl
from jax.experimental.pallas import tpu as pltpu
from jax.experimental.pallas import tpu_sc as plsc

ROWS = 16384
COLS = 16384
W = 164
NNZ = ROWS * W

NUM_WORKERS = 32
RPW = ROWS // NUM_WORKERS
CHUNK_ROWS = 128
CHUNK_NNZ = CHUNK_ROWS * W
N_CHUNKS = RPW // CHUNK_ROWS
STRIPS = CHUNK_ROWS // 16


def _spmv_body(x_hbm, vals_hbm, cols_hbm, y_hbm, x_v, vals_v0, vals_v1,
               cols_v0, cols_v1, y_v, x_sem, v_sems, c_sems):
    wid = lax.axis_index("s") * 2 + lax.axis_index("c")
    base_row = wid * RPW
    vals_bufs = (vals_v0, vals_v1)
    cols_bufs = (cols_v0, cols_v1)

    x_cp = pltpu.async_copy(x_hbm, x_v, x_sem)

    def start_chunk(c):
        nnz0 = (base_row + c * CHUNK_ROWS) * W
        b = c % 2
        vcp = pltpu.async_copy(
            vals_hbm.at[pl.ds(nnz0, CHUNK_NNZ)], vals_bufs[b], v_sems.at[b])
        ccp = pltpu.async_copy(
            cols_hbm.at[pl.ds(nnz0, CHUNK_NNZ)], cols_bufs[b], c_sems.at[b])
        return vcp, ccp

    lane = lax.broadcasted_iota(jnp.int32, (16,), 0)
    stride_idx = lane * W

    cps = start_chunk(0)
    x_cp.wait()

    for c in range(N_CHUNKS):
        nxt = start_chunk(c + 1) if c + 1 < N_CHUNKS else None
        cps[0].wait()
        cps[1].wait()
        b = c % 2
        vals_b = vals_bufs[b]
        cols_b = cols_bufs[b]

        @plsc.parallel_loop(0, STRIPS, step=1)
        def strip_body(s):
            base = s * (16 * W)
            zero = jnp.zeros((16,), jnp.float32)

            @plsc.parallel_loop(0, W, step=4, carry=(zero, zero, zero, zero))
            def j_loop(j, accs):
                outs = []
                for u in range(4):
                    jw = lane + (j + u)
                    jw = jnp.where(jw >= W, jw - W, jw)
                    pos = stride_idx + base + jw
                    cv = plsc.load_gather(cols_b, [pos])
                    vv = plsc.load_gather(vals_b, [pos])
                    xg = plsc.load_gather(x_v, [cv])
                    outs.append(accs[u] + vv * xg)
                return tuple(outs)

            a0, a1, a2, a3 = j_loop
            y_v[pl.ds(c * CHUNK_ROWS + s * 16, 16)] = (a0 + a1) + (a2 + a3)

        cps = nxt

    pltpu.sync_copy(y_v, y_hbm.at[pl.ds(wid * RPW, RPW)])


@jax.jit
def _spmv(x, values, cols_i32):
    mesh = plsc.VectorSubcoreMesh(core_axis_name="c", subcore_axis_name="s")
    return pl.kernel(
        _spmv_body,
        mesh=mesh,
        compiler_params=pltpu.CompilerParams(needs_layout_passes=False),
        out_type=jax.ShapeDtypeStruct((ROWS,), jnp.float32),
        scratch_types=[
            pltpu.VMEM((COLS,), jnp.float32),
            pltpu.VMEM((CHUNK_NNZ,), jnp.float32),
            pltpu.VMEM((CHUNK_NNZ,), jnp.float32),
            pltpu.VMEM((CHUNK_NNZ,), jnp.int32),
            pltpu.VMEM((CHUNK_NNZ,), jnp.int32),
            pltpu.VMEM((RPW,), jnp.float32),
            pltpu.SemaphoreType.DMA,
            pltpu.SemaphoreType.DMA((2,)),
            pltpu.SemaphoreType.DMA((2,)),
        ],
    )(x, values, cols_i32)


def kernel(x, values, col_indices, row_ptrs):
    del row_ptrs
    return _spmv(x, values, col_indices.astype(jnp.int32))
